# baseline (speedup 1.0000x reference)
import functools

import jax
import jax.numpy as jnp
from jax.experimental import pallas as pl
from jax.experimental.pallas import tpu as pltpu

_B, _S, _D, _E = 4, 8192, 768, 64
_CHUNK = 2048


def _router_body(x_ref, w_ref, b_ref, noise_ref, comb_ref, idx_ref, sc_ref,
                 acc_ref):
    r = pl.program_id(0)
    nr = pl.num_programs(0)

    @pl.when(r == 0)
    def _init():
        acc_ref[...] = jnp.zeros_like(acc_ref)

    acc_ref[...] += jnp.sum(x_ref[...], axis=1)

    @pl.when(r == nr - 1)
    def _epilogue():
        mean = acc_ref[...] * (1.0 / _S)
        scores = jnp.dot(mean, w_ref[...],
                         preferred_element_type=jnp.float32)
        scores = scores + b_ref[...][None, :] + noise_ref[...]
        m = jnp.max(scores, axis=-1, keepdims=True)
        ex = jnp.exp(scores - m)
        gates = ex / jnp.sum(ex, axis=-1, keepdims=True)
        iota = jax.lax.broadcasted_iota(jnp.int32, (_B, _E), 1)
        s1 = jnp.max(gates, axis=-1, keepdims=True)
        i1 = jnp.min(jnp.where(gates == s1, iota, _E), axis=-1, keepdims=True)
        masked = jnp.where(iota == i1, -jnp.inf, gates)
        s2 = jnp.max(masked, axis=-1, keepdims=True)
        i2 = jnp.min(jnp.where(masked == s2, iota, _E), axis=-1, keepdims=True)
        denom = s1 + s2 + 1e-9
        comb_ref[...] = (jnp.where(iota == i1, s1 / denom, 0.0)
                         + jnp.where(iota == i2, s2 / denom, 0.0))
        idx_ref[...] = jnp.concatenate([i1, i2], axis=1)
        sc_ref[...] = jnp.concatenate([s1, s2], axis=1)


@jax.jit
def kernel(x, W_l3, b_l3, noise):
    nr = _S // _CHUNK
    comb, idx, scores = pl.pallas_call(
        _router_body,
        grid=(nr,),
        in_specs=[
            pl.BlockSpec((_B, _CHUNK, _D), lambda r: (0, r, 0)),
            pl.BlockSpec((_D, _E), lambda r: (0, 0)),
            pl.BlockSpec((_E,), lambda r: (0,)),
            pl.BlockSpec((_B, _E), lambda r: (0, 0)),
        ],
        out_specs=[
            pl.BlockSpec((_B, _E), lambda r: (0, 0)),
            pl.BlockSpec((_B, 2), lambda r: (0, 0)),
            pl.BlockSpec((_B, 2), lambda r: (0, 0)),
        ],
        out_shape=[
            jax.ShapeDtypeStruct((_B, _E), jnp.float32),
            jax.ShapeDtypeStruct((_B, 2), jnp.int32),
            jax.ShapeDtypeStruct((_B, 2), jnp.float32),
        ],
        scratch_shapes=[pltpu.VMEM((_B, _D), jnp.float32)],
        compiler_params=pltpu.CompilerParams(
            dimension_semantics=("arbitrary",)),
    )(x, W_l3, b_l3, noise)
    return comb, idx, scores


# speedup vs baseline: 1.0259x; 1.0259x over previous
import functools

import jax
import jax.numpy as jnp
from jax.experimental import pallas as pl
from jax.experimental.pallas import tpu as pltpu

_B, _S, _D, _E = 4, 8192, 768, 64
_CHUNK = 512


def _router_body(x_ref, w_ref, b_ref, noise_ref, comb_ref, idx_ref, sc_ref,
                 acc_ref):
    r = pl.program_id(0)
    nr = pl.num_programs(0)

    @pl.when(r == 0)
    def _init():
        acc_ref[...] = jnp.zeros_like(acc_ref)

    acc_ref[...] += jnp.sum(x_ref[...], axis=1)

    @pl.when(r == nr - 1)
    def _epilogue():
        mean = acc_ref[...] * (1.0 / _S)
        scores = jnp.dot(mean, w_ref[...],
                         preferred_element_type=jnp.float32)
        scores = scores + b_ref[...][None, :] + noise_ref[...]
        m = jnp.max(scores, axis=-1, keepdims=True)
        ex = jnp.exp(scores - m)
        gates = ex / jnp.sum(ex, axis=-1, keepdims=True)
        iota = jax.lax.broadcasted_iota(jnp.int32, (_B, _E), 1)
        s1 = jnp.max(gates, axis=-1, keepdims=True)
        i1 = jnp.min(jnp.where(gates == s1, iota, _E), axis=-1, keepdims=True)
        masked = jnp.where(iota == i1, -jnp.inf, gates)
        s2 = jnp.max(masked, axis=-1, keepdims=True)
        i2 = jnp.min(jnp.where(masked == s2, iota, _E), axis=-1, keepdims=True)
        denom = s1 + s2 + 1e-9
        comb_ref[...] = (jnp.where(iota == i1, s1 / denom, 0.0)
                         + jnp.where(iota == i2, s2 / denom, 0.0))
        idx_ref[...] = jnp.concatenate([i1, i2], axis=1)
        sc_ref[...] = jnp.concatenate([s1, s2], axis=1)


@jax.jit
def kernel(x, W_l3, b_l3, noise):
    nr = _S // _CHUNK
    comb, idx, scores = pl.pallas_call(
        _router_body,
        grid=(nr,),
        in_specs=[
            pl.BlockSpec((_B, _CHUNK, _D), lambda r: (0, r, 0)),
            pl.BlockSpec((_D, _E), lambda r: (0, 0)),
            pl.BlockSpec((_E,), lambda r: (0,)),
            pl.BlockSpec((_B, _E), lambda r: (0, 0)),
        ],
        out_specs=[
            pl.BlockSpec((_B, _E), lambda r: (0, 0)),
            pl.BlockSpec((_B, 2), lambda r: (0, 0)),
            pl.BlockSpec((_B, 2), lambda r: (0, 0)),
        ],
        out_shape=[
            jax.ShapeDtypeStruct((_B, _E), jnp.float32),
            jax.ShapeDtypeStruct((_B, 2), jnp.int32),
            jax.ShapeDtypeStruct((_B, 2), jnp.float32),
        ],
        scratch_shapes=[pltpu.VMEM((_B, _D), jnp.float32)],
        compiler_params=pltpu.CompilerParams(
            dimension_semantics=("arbitrary",)),
    )(x, W_l3, b_l3, noise)
    return comb, idx, scores
